# Initial kernel scaffold; baseline (speedup 1.0000x reference)
import jax
import jax.numpy as jnp
from jax.experimental import pallas as pl


def kernel(xyz1, xyz2, points1, points2, W0, b0, scale0, bias0, mean0, var0, W1, b1, scale1, bias1, mean1, var1):
    raise NotImplementedError("write your pallas kernel here")


# speedup vs baseline: 38.1372x; 38.1372x over previous
import jax
import jax.numpy as jnp
from jax.experimental import pallas as pl

N_PTS = 16384
S_PTS = 4096
D1 = 128
D2 = 256
H0 = 256
H1 = 128
BQ = 128
BIG = 3.0e38


def _fp_body(x1b_ref, x2_ref, p1b_ref, p2t_ref, w0_ref, b0_ref, w1_ref, b1_ref,
             out_ref):
    x1b = x1b_ref[...]
    x2 = x2_ref[...]
    mm = jnp.dot(x1b * -2.0, x2, preferred_element_type=jnp.float32)
    n1 = jnp.sum(x1b * x1b, axis=1, keepdims=True)
    n2 = jnp.sum(x2 * x2, axis=0, keepdims=True)
    D = mm + n1 + n2

    lane = jax.lax.broadcasted_iota(jnp.int32, (BQ, S_PTS), 1)
    ds = []
    idxs = []
    for _ in range(3):
        m = jnp.min(D, axis=1, keepdims=True)
        idx = jnp.min(jnp.where(D == m, lane, S_PTS), axis=1, keepdims=True)
        ds.append(m)
        idxs.append(idx)
        D = jnp.where(lane == idx, BIG, D)

    r = [1.0 / (d + 1e-8) for d in ds]
    norm = r[0] + r[1] + r[2]
    w = [ri / norm for ri in r]
    wmat = (jnp.where(lane == idxs[0], w[0], 0.0)
            + jnp.where(lane == idxs[1], w[1], 0.0)
            + jnp.where(lane == idxs[2], w[2], 0.0))
    interp = jnp.dot(wmat, p2t_ref[...], preferred_element_type=jnp.float32)

    x = jnp.concatenate([p1b_ref[...], interp], axis=1)
    h = jnp.dot(x, w0_ref[...], preferred_element_type=jnp.float32) + b0_ref[...]
    h = jnp.maximum(h, 0.0)
    h = jnp.dot(h, w1_ref[...], preferred_element_type=jnp.float32) + b1_ref[...]
    h = jnp.maximum(h, 0.0)
    out_ref[...] = h.T


def kernel(xyz1, xyz2, points1, points2, W0, b0, scale0, bias0, mean0, var0,
           W1, b1, scale1, bias1, mean1, var1):
    eps = 1e-5
    a0 = scale0 / jnp.sqrt(var0 + eps)
    W0f = W0 * a0[None, :]
    b0f = ((b0 - mean0) * a0 + bias0).reshape(1, H0)
    a1 = scale1 / jnp.sqrt(var1 + eps)
    W1f = W1 * a1[None, :]
    b1f = ((b1 - mean1) * a1 + bias1).reshape(1, H1)

    x1p = jnp.pad(xyz1.T, ((0, 0), (0, 5)))
    x2p = jnp.pad(xyz2, ((0, 5), (0, 0)))
    p1t = points1.T
    p2t = points2.T

    grid = (N_PTS // BQ,)
    out = pl.pallas_call(
        _fp_body,
        grid=grid,
        in_specs=[
            pl.BlockSpec((BQ, 8), lambda i: (i, 0)),
            pl.BlockSpec((8, S_PTS), lambda i: (0, 0)),
            pl.BlockSpec((BQ, D1), lambda i: (i, 0)),
            pl.BlockSpec((S_PTS, D2), lambda i: (0, 0)),
            pl.BlockSpec((D1 + D2, H0), lambda i: (0, 0)),
            pl.BlockSpec((1, H0), lambda i: (0, 0)),
            pl.BlockSpec((H0, H1), lambda i: (0, 0)),
            pl.BlockSpec((1, H1), lambda i: (0, 0)),
        ],
        out_specs=pl.BlockSpec((H1, BQ), lambda i: (0, i)),
        out_shape=jax.ShapeDtypeStruct((H1, N_PTS), jnp.float32),
    )(x1p, x2p, p1t, p2t, W0f, b0f, W1f, b1f)
    return out


# speedup vs baseline: 53.7786x; 1.4101x over previous
import jax
import jax.numpy as jnp
from jax.experimental import pallas as pl

N_PTS = 16384
S_PTS = 4096
D1 = 128
D2 = 256
H0 = 256
H1 = 128
BQ = 128
BIG = 3.0e38


def _fp_body(x1b_ref, x2_ref, p1b_ref, p2t_ref, w0_ref, b0_ref, w1_ref, b1_ref,
             out_ref):
    x1b = x1b_ref[...]
    x2 = x2_ref[...]
    n2 = jnp.sum(x2 * x2, axis=0, keepdims=True)
    mm = jnp.dot(x1b * -2.0, x2, preferred_element_type=jnp.float32)
    n1 = jnp.sum(x1b * x1b, axis=1, keepdims=True)
    D = mm + n1 + n2

    rs = []
    wmat_u = None
    for k in range(3):
        m = jnp.min(D, axis=1, keepdims=True)
        match = D == m
        r = 1.0 / (m + 1e-8)
        rs.append(r)
        hit = jnp.where(match, r, 0.0)
        wmat_u = hit if wmat_u is None else wmat_u + hit
        if k < 2:
            D = jnp.where(match, BIG, D)

    inv_norm = 1.0 / (rs[0] + rs[1] + rs[2])
    interp = jnp.dot(wmat_u, p2t_ref[...],
                     preferred_element_type=jnp.float32) * inv_norm

    x = jnp.concatenate([p1b_ref[...], interp], axis=1)
    h = jnp.dot(x, w0_ref[...], preferred_element_type=jnp.float32) + b0_ref[...]
    h = jnp.maximum(h, 0.0)
    h = jnp.dot(h, w1_ref[...], preferred_element_type=jnp.float32) + b1_ref[...]
    h = jnp.maximum(h, 0.0)
    out_ref[...] = h.T


def kernel(xyz1, xyz2, points1, points2, W0, b0, scale0, bias0, mean0, var0,
           W1, b1, scale1, bias1, mean1, var1):
    eps = 1e-5
    a0 = scale0 / jnp.sqrt(var0 + eps)
    W0f = W0 * a0[None, :]
    b0f = ((b0 - mean0) * a0 + bias0).reshape(1, H0)
    a1 = scale1 / jnp.sqrt(var1 + eps)
    W1f = W1 * a1[None, :]
    b1f = ((b1 - mean1) * a1 + bias1).reshape(1, H1)

    x1p = jnp.pad(xyz1.T, ((0, 0), (0, 5)))
    x2p = jnp.pad(xyz2, ((0, 5), (0, 0)))
    p1t = points1.T
    p2t = points2.T

    grid = (N_PTS // BQ,)
    out = pl.pallas_call(
        _fp_body,
        grid=grid,
        in_specs=[
            pl.BlockSpec((BQ, 8), lambda i: (i, 0)),
            pl.BlockSpec((8, S_PTS), lambda i: (0, 0)),
            pl.BlockSpec((BQ, D1), lambda i: (i, 0)),
            pl.BlockSpec((S_PTS, D2), lambda i: (0, 0)),
            pl.BlockSpec((D1 + D2, H0), lambda i: (0, 0)),
            pl.BlockSpec((1, H0), lambda i: (0, 0)),
            pl.BlockSpec((H0, H1), lambda i: (0, 0)),
            pl.BlockSpec((1, H1), lambda i: (0, 0)),
        ],
        out_specs=pl.BlockSpec((H1, BQ), lambda i: (0, i)),
        out_shape=jax.ShapeDtypeStruct((H1, N_PTS), jnp.float32),
    )(x1p, x2p, p1t, p2t, W0f, b0f, W1f, b1f)
    return out


# speedup vs baseline: 63.8946x; 1.1881x over previous
import jax
import jax.numpy as jnp
from jax.experimental import pallas as pl

N_PTS = 16384
S_PTS = 4096
D1 = 128
D2 = 256
H0 = 256
H1 = 128
BQ = 256
BIG = 3.0e38


def _fp_body(x1b_ref, x2_ref, p1b_ref, p2t_ref, w0_ref, b0_ref, w1_ref, b1_ref,
             out_ref):
    x1b = x1b_ref[...]
    x2 = x2_ref[...]
    n2 = jnp.sum(x2 * x2, axis=0, keepdims=True)
    mm = jnp.dot(x1b * -2.0, x2, preferred_element_type=jnp.float32)
    n1 = jnp.sum(x1b * x1b, axis=1, keepdims=True)
    D = mm + n2
    rs = []
    wmat_u = None
    for k in range(3):
        m = jnp.min(D, axis=1, keepdims=True)
        match = D == m
        r = 1.0 / (m + n1 + 1e-8)
        rs.append(r)
        hit = jnp.where(match, r, 0.0)
        wmat_u = hit if wmat_u is None else wmat_u + hit
        if k < 2:
            D = jnp.where(match, BIG, D)

    inv_norm = 1.0 / (rs[0] + rs[1] + rs[2])
    interp = jnp.dot(wmat_u, p2t_ref[...],
                     preferred_element_type=jnp.float32) * inv_norm

    x = jnp.concatenate([p1b_ref[...], interp], axis=1)
    h = jnp.dot(x, w0_ref[...], preferred_element_type=jnp.float32) + b0_ref[...]
    h = jnp.maximum(h, 0.0)
    h = jnp.dot(h, w1_ref[...], preferred_element_type=jnp.float32) + b1_ref[...]
    h = jnp.maximum(h, 0.0)
    out_ref[...] = h.T


def kernel(xyz1, xyz2, points1, points2, W0, b0, scale0, bias0, mean0, var0,
           W1, b1, scale1, bias1, mean1, var1):
    eps = 1e-5
    a0 = scale0 / jnp.sqrt(var0 + eps)
    W0f = W0 * a0[None, :]
    b0f = ((b0 - mean0) * a0 + bias0).reshape(1, H0)
    a1 = scale1 / jnp.sqrt(var1 + eps)
    W1f = W1 * a1[None, :]
    b1f = ((b1 - mean1) * a1 + bias1).reshape(1, H1)

    x1p = jnp.pad(xyz1.T, ((0, 0), (0, 5)))
    x2p = jnp.pad(xyz2, ((0, 5), (0, 0)))
    p1t = points1.T
    p2t = points2.T

    grid = (N_PTS // BQ,)
    out = pl.pallas_call(
        _fp_body,
        grid=grid,
        in_specs=[
            pl.BlockSpec((BQ, 8), lambda i: (i, 0)),
            pl.BlockSpec((8, S_PTS), lambda i: (0, 0)),
            pl.BlockSpec((BQ, D1), lambda i: (i, 0)),
            pl.BlockSpec((S_PTS, D2), lambda i: (0, 0)),
            pl.BlockSpec((D1 + D2, H0), lambda i: (0, 0)),
            pl.BlockSpec((1, H0), lambda i: (0, 0)),
            pl.BlockSpec((H0, H1), lambda i: (0, 0)),
            pl.BlockSpec((1, H1), lambda i: (0, 0)),
        ],
        out_specs=pl.BlockSpec((H1, BQ), lambda i: (0, i)),
        out_shape=jax.ShapeDtypeStruct((H1, N_PTS), jnp.float32),
    )(x1p, x2p, p1t, p2t, W0f, b0f, W1f, b1f)
    return out


# speedup vs baseline: 65.7533x; 1.0291x over previous
import jax
import jax.numpy as jnp
from jax.experimental import pallas as pl
from jax.experimental.pallas import tpu as pltpu

N_PTS = 16384
S_PTS = 4096
D1 = 128
D2 = 256
H0 = 256
H1 = 128
BQ = 256
NB = N_PTS // BQ
BIG = 3.0e38


def _fp_body(x1b_ref, x2_ref, p1b_ref, p2t_ref, w0_ref, b0_ref, w1_ref,
             b1_ref, out_ref, dscr, n1scr):
    i = pl.program_id(0)
    par = i % 2

    @pl.when(i < NB)
    def _produce():
        x1b = x1b_ref[...]
        x2 = x2_ref[...]
        n2 = jnp.sum(x2 * x2, axis=0, keepdims=True)
        mm = jnp.dot(x1b * -2.0, x2, preferred_element_type=jnp.float32)
        dscr[par] = mm + n2
        n1scr[par] = jnp.sum(x1b * x1b, axis=1, keepdims=True)

    @pl.when(i > 0)
    def _consume():
        D = dscr[1 - par]
        n1 = n1scr[1 - par]

        rs = []
        wmat_u = None
        for k in range(3):
            m = jnp.min(D, axis=1, keepdims=True)
            match = D == m
            r = 1.0 / (m + n1 + 1e-8)
            rs.append(r)
            hit = jnp.where(match, r, 0.0)
            wmat_u = hit if wmat_u is None else wmat_u + hit
            if k < 2:
                D = jnp.where(match, BIG, D)

        inv_norm = 1.0 / (rs[0] + rs[1] + rs[2])
        interp = jnp.dot(wmat_u, p2t_ref[...],
                         preferred_element_type=jnp.float32) * inv_norm

        x = jnp.concatenate([p1b_ref[...], interp], axis=1)
        h = jnp.dot(x, w0_ref[...],
                    preferred_element_type=jnp.float32) + b0_ref[...]
        h = jnp.maximum(h, 0.0)
        h = jnp.dot(h, w1_ref[...],
                    preferred_element_type=jnp.float32) + b1_ref[...]
        h = jnp.maximum(h, 0.0)
        out_ref[...] = h.T


def kernel(xyz1, xyz2, points1, points2, W0, b0, scale0, bias0, mean0, var0,
           W1, b1, scale1, bias1, mean1, var1):
    eps = 1e-5
    a0 = scale0 / jnp.sqrt(var0 + eps)
    W0f = W0 * a0[None, :]
    b0f = ((b0 - mean0) * a0 + bias0).reshape(1, H0)
    a1 = scale1 / jnp.sqrt(var1 + eps)
    W1f = W1 * a1[None, :]
    b1f = ((b1 - mean1) * a1 + bias1).reshape(1, H1)

    x1p = jnp.pad(xyz1.T, ((0, 0), (0, 5)))
    x2p = jnp.pad(xyz2, ((0, 5), (0, 0)))
    p1t = points1.T
    p2t = points2.T

    grid = (NB + 1,)
    out = pl.pallas_call(
        _fp_body,
        grid=grid,
        in_specs=[
            pl.BlockSpec((BQ, 8), lambda i: (jnp.minimum(i, NB - 1), 0)),
            pl.BlockSpec((8, S_PTS), lambda i: (0, 0)),
            pl.BlockSpec((BQ, D1), lambda i: (jnp.maximum(i - 1, 0), 0)),
            pl.BlockSpec((S_PTS, D2), lambda i: (0, 0)),
            pl.BlockSpec((D1 + D2, H0), lambda i: (0, 0)),
            pl.BlockSpec((1, H0), lambda i: (0, 0)),
            pl.BlockSpec((H0, H1), lambda i: (0, 0)),
            pl.BlockSpec((1, H1), lambda i: (0, 0)),
        ],
        out_specs=pl.BlockSpec((H1, BQ), lambda i: (0, jnp.maximum(i - 1, 0))),
        out_shape=jax.ShapeDtypeStruct((H1, N_PTS), jnp.float32),
        scratch_shapes=[
            pltpu.VMEM((2, BQ, S_PTS), jnp.float32),
            pltpu.VMEM((2, BQ, 1), jnp.float32),
        ],
    )(x1p, x2p, p1t, p2t, W0f, b0f, W1f, b1f)
    return out


# speedup vs baseline: 69.9824x; 1.0643x over previous
import jax
import jax.numpy as jnp
from jax.experimental import pallas as pl
from jax.experimental.pallas import tpu as pltpu

N_PTS = 16384
S_PTS = 4096
D1 = 128
D2 = 256
H0 = 256
H1 = 128
BQ = 256
NB = N_PTS // BQ
BIG = 3.0e38


def _produce(x1b_ref, x2_ref, dref, n1ref):
    x1b = x1b_ref[...]
    x2 = x2_ref[...]
    n2 = jnp.sum(x2 * x2, axis=0, keepdims=True)
    mm = jnp.dot(x1b * -2.0, x2, preferred_element_type=jnp.float32)
    dref[...] = mm + n2
    n1ref[...] = jnp.sum(x1b * x1b, axis=1, keepdims=True)


def _consume(dref, n1ref, p1b_ref, p2t_ref, w0_ref, b0_ref, w1_ref, b1_ref,
             out_ref):
    D = dref[...]
    n1 = n1ref[...]

    m1 = jnp.min(D, axis=1, keepdims=True)
    Dm = jnp.where(D == m1, BIG, D)
    m2 = jnp.min(Dm, axis=1, keepdims=True)
    m3 = jnp.min(jnp.where(Dm == m2, BIG, Dm), axis=1, keepdims=True)

    r1 = 1.0 / (m1 + n1 + 1e-8)
    r2 = 1.0 / (m2 + n1 + 1e-8)
    r3 = 1.0 / (m3 + n1 + 1e-8)
    wmat_u = jnp.where(D == m1, r1,
                       jnp.where(D == m2, r2,
                                 jnp.where(D == m3, r3, 0.0)))

    inv_norm = 1.0 / (r1 + r2 + r3)
    interp = jnp.dot(wmat_u, p2t_ref[...],
                     preferred_element_type=jnp.float32) * inv_norm

    x = jnp.concatenate([p1b_ref[...], interp], axis=1)
    h = jnp.dot(x, w0_ref[...],
                preferred_element_type=jnp.float32) + b0_ref[...]
    h = jnp.maximum(h, 0.0)
    h = jnp.dot(h, w1_ref[...],
                preferred_element_type=jnp.float32) + b1_ref[...]
    h = jnp.maximum(h, 0.0)
    out_ref[...] = h.T


def _fp_body(x1b_ref, x2_ref, p1b_ref, p2t_ref, w0_ref, b0_ref, w1_ref,
             b1_ref, out_ref, dscr0, dscr1, n1scr0, n1scr1):
    i = pl.program_id(0)
    par = i % 2

    @pl.when((i < NB) & (par == 0))
    def _():
        _produce(x1b_ref, x2_ref, dscr0, n1scr0)

    @pl.when((i < NB) & (par == 1))
    def _():
        _produce(x1b_ref, x2_ref, dscr1, n1scr1)

    @pl.when((i > 0) & (par == 1))
    def _():
        _consume(dscr0, n1scr0, p1b_ref, p2t_ref, w0_ref, b0_ref, w1_ref,
                 b1_ref, out_ref)

    @pl.when((i > 0) & (par == 0))
    def _():
        _consume(dscr1, n1scr1, p1b_ref, p2t_ref, w0_ref, b0_ref, w1_ref,
                 b1_ref, out_ref)


def kernel(xyz1, xyz2, points1, points2, W0, b0, scale0, bias0, mean0, var0,
           W1, b1, scale1, bias1, mean1, var1):
    eps = 1e-5
    a0 = scale0 / jnp.sqrt(var0 + eps)
    W0f = W0 * a0[None, :]
    b0f = ((b0 - mean0) * a0 + bias0).reshape(1, H0)
    a1 = scale1 / jnp.sqrt(var1 + eps)
    W1f = W1 * a1[None, :]
    b1f = ((b1 - mean1) * a1 + bias1).reshape(1, H1)

    x1p = jnp.pad(xyz1.T, ((0, 0), (0, 5)))
    x2p = jnp.pad(xyz2, ((0, 5), (0, 0)))
    p1t = points1.T
    p2t = points2.T

    grid = (NB + 1,)
    out = pl.pallas_call(
        _fp_body,
        grid=grid,
        in_specs=[
            pl.BlockSpec((BQ, 8), lambda i: (jnp.minimum(i, NB - 1), 0)),
            pl.BlockSpec((8, S_PTS), lambda i: (0, 0)),
            pl.BlockSpec((BQ, D1), lambda i: (jnp.maximum(i - 1, 0), 0)),
            pl.BlockSpec((S_PTS, D2), lambda i: (0, 0)),
            pl.BlockSpec((D1 + D2, H0), lambda i: (0, 0)),
            pl.BlockSpec((1, H0), lambda i: (0, 0)),
            pl.BlockSpec((H0, H1), lambda i: (0, 0)),
            pl.BlockSpec((1, H1), lambda i: (0, 0)),
        ],
        out_specs=pl.BlockSpec((H1, BQ), lambda i: (0, jnp.maximum(i - 1, 0))),
        out_shape=jax.ShapeDtypeStruct((H1, N_PTS), jnp.float32),
        scratch_shapes=[
            pltpu.VMEM((BQ, S_PTS), jnp.float32),
            pltpu.VMEM((BQ, S_PTS), jnp.float32),
            pltpu.VMEM((BQ, 1), jnp.float32),
            pltpu.VMEM((BQ, 1), jnp.float32),
        ],
    )(x1p, x2p, p1t, p2t, W0f, b0f, W1f, b1f)
    return out
